# baseline (speedup 1.0000x reference)
import functools

import jax
import jax.numpy as jnp
from jax import lax
from jax.experimental import pallas as pl
from jax.experimental.pallas import tpu as pltpu
from jax.experimental.pallas import tpu_sc as plsc

_RB = 8192
_WR = 2048


def _flatroll(x, s):
    sl = jnp.mod(s, 128)
    sr = s // 128
    xr = pltpu.roll(x, sl, axis=1)
    y0 = pltpu.roll(xr, sr, axis=0)
    y1 = pltpu.roll(xr, sr + 1, axis=0)
    col = lax.broadcasted_iota(jnp.int32, x.shape, 1)
    return jnp.where(col < sl, y1, y0)


def _emb_body(s_ref, emb, qe, se, ne, er_s):
    eb = _RB * 128
    ce = pl.num_programs(0) * eb
    wl = _WR * 128
    w0 = s_ref[0]
    g = pl.program_id(0)

    @pl.when(g == 0)
    def _():
        er_s[...] = _flatroll(emb[...], jnp.mod(w0, wl))

    se[...] = qe[...]
    t0 = jnp.mod(g * eb - w0, ce)
    er = er_s[...]
    for k in range(_RB // _WR):
        fi = (lax.broadcasted_iota(jnp.int32, (_WR, 128), 0) * 128
              + lax.broadcasted_iota(jnp.int32, (_WR, 128), 1))
        tt = t0 + k * (_WR * 128) + fi
        tt = jnp.where(tt >= ce, tt - ce, tt)
        mask = tt < wl
        ne[k * _WR:(k + 1) * _WR, :] = jnp.where(mask, er, qe[k * _WR:(k + 1) * _WR, :])


def _make_ids_sc_kernel(C, B):
    NC = 16
    SL = C // NC
    BW = B // NC
    KS = 128
    NK = BW // KS
    mesh = plsc.VectorSubcoreMesh(core_axis_name="c", subcore_axis_name="s")

    @functools.partial(
        pl.kernel,
        out_type=[
            jax.ShapeDtypeStruct((C,), jnp.int32),
            jax.ShapeDtypeStruct((C,), jnp.int32),
            jax.ShapeDtypeStruct((C,), jnp.int32),
            jax.ShapeDtypeStruct((C,), jnp.int32),
        ],
        mesh=mesh,
        scratch_types=[
            pltpu.VMEM((SL,), jnp.int32),
            pltpu.VMEM((SL,), jnp.int32),
            [pltpu.VMEM((KS,), jnp.int32) for _ in range(NK)],
            [pltpu.VMEM((KS,), jnp.int32) for _ in range(NK)],
            [pltpu.VMEM((KS,), jnp.int32) for _ in range(NK)],
            pltpu.SemaphoreType.DMA,
        ],
    )
    def ids_k(idx_hbm, ilo_hbm, ihi_hbm, qlo_hbm, qhi_hbm,
              slo_hbm, shi_hbm, nlo_hbm, nhi_hbm,
              va, vb, vidx, vlo, vhi, sem):
        c = lax.axis_index("c")
        s = lax.axis_index("s")
        base = s * SL

        @pl.when(c == 1)
        def _():
            pltpu.sync_copy(qlo_hbm.at[pl.ds(base, SL)], va)
            pltpu.sync_copy(va, slo_hbm.at[pl.ds(base, SL)])
            pltpu.sync_copy(qhi_hbm.at[pl.ds(base, SL)], vb)
            pltpu.sync_copy(vb, shi_hbm.at[pl.ds(base, SL)])

        @pl.when(c == 0)
        def _():
            pltpu.sync_copy(qlo_hbm.at[pl.ds(base, SL)], va)
            pltpu.sync_copy(va, nlo_hbm.at[pl.ds(base, SL)])
            pltpu.sync_copy(qhi_hbm.at[pl.ds(base, SL)], vb)
            pltpu.sync_copy(vb, nhi_hbm.at[pl.ds(base, SL)])
            plsc.subcore_barrier()
            for k in range(NK):
                off = s * BW + k * KS
                pltpu.sync_copy(idx_hbm.at[pl.ds(off, KS)], vidx[k])
                pltpu.sync_copy(ilo_hbm.at[pl.ds(off, KS)], vlo[k])
                pltpu.sync_copy(ihi_hbm.at[pl.ds(off, KS)], vhi[k])
                pltpu.async_copy(vlo[k], nlo_hbm.at[vidx[k]], sem).wait()
                pltpu.async_copy(vhi[k], nhi_hbm.at[vidx[k]], sem).wait()

    return ids_k


def kernel(embeddings, item_ids, queue_embeddings, queue_item_ids, ptr):
    C, D = queue_embeddings.shape
    B = embeddings.shape[0]
    p = jnp.asarray(jnp.mod(ptr, C), jnp.int32)

    CE = C * D
    G = CE // (_RB * 128)
    emb2 = embeddings.reshape(_WR, 128)
    qe2 = queue_embeddings.reshape(CE // 128, 128)
    scal = jnp.stack([D * p, jnp.int32(0)])

    se2, ne2 = pl.pallas_call(
        _emb_body,
        grid_spec=pltpu.PrefetchScalarGridSpec(
            num_scalar_prefetch=1,
            grid=(G,),
            in_specs=[
                pl.BlockSpec((_WR, 128), lambda g, pr: (jnp.int32(0), jnp.int32(0))),
                pl.BlockSpec((_RB, 128), lambda g, pr: (g, jnp.int32(0))),
            ],
            out_specs=[
                pl.BlockSpec((_RB, 128), lambda g, pr: (g, jnp.int32(0))),
                pl.BlockSpec((_RB, 128), lambda g, pr: (g, jnp.int32(0))),
            ],
            scratch_shapes=[pltpu.VMEM((_WR, 128), jnp.float32)],
        ),
        out_shape=[
            jax.ShapeDtypeStruct((CE // 128, 128), jnp.float32),
            jax.ShapeDtypeStruct((CE // 128, 128), jnp.float32),
        ],
    )(scal, emb2, qe2)

    ilo = item_ids.astype(jnp.int32)
    ihi = jnp.right_shift(item_ids, 32).astype(jnp.int32)
    qlo = queue_item_ids.astype(jnp.int32)
    qhi = jnp.right_shift(queue_item_ids, 32).astype(jnp.int32)
    idx_arr = jnp.mod(p + jnp.arange(B, dtype=jnp.int32), C).astype(jnp.int32)

    ids_k = _make_ids_sc_kernel(C, B)
    slo, shi, nlo, nhi = ids_k(idx_arr, ilo, ihi, qlo, qhi)

    def _to64(hi, lo):
        return (jnp.left_shift(hi.reshape(C).astype(jnp.int64), 32)
                | (lo.reshape(C).astype(jnp.int64) & jnp.int64(0xFFFFFFFF)))

    se = se2.reshape(C, D)
    ne = ne2.reshape(C, D)
    return (se, _to64(shi, slo), ne, _to64(nhi, nlo))
